# Initial kernel scaffold; baseline (speedup 1.0000x reference)
import jax
import jax.numpy as jnp
from jax.experimental import pallas as pl


def kernel(x, edge_index, W1, b1, Wg, att_src, att_dst, bg, W2, b2):
    raise NotImplementedError("write your pallas kernel here")


# speedup vs baseline: 31.2271x; 31.2271x over previous
import functools

import jax
import jax.numpy as jnp
from jax import lax
from jax.experimental import pallas as pl
from jax.experimental.pallas import tpu as pltpu
from jax.experimental.pallas import tpu_sc as plsc

NC = 2
NS = 16
NL = 16
NW = NC * NS
K = 128


def _mesh():
    return plsc.VectorSubcoreMesh(core_axis_name="c", subcore_axis_name="s")


ZR = 8


def _zero_rows(zbuf, nrow, ncol):
    def zrow(j, _):
        for t in range(ncol // NL):
            zbuf[j, pl.ds(t * NL, NL)] = jnp.zeros((NL,), jnp.float32)
        return 0
    lax.fori_loop(0, nrow, zrow, 0)


def _zero_shared(acc, zbuf, s, nt, ncol):
    zr = zbuf.shape[0]
    rows = nt // NS
    base = s * rows
    def zc(i, _):
        pltpu.sync_copy(zbuf, acc.at[pl.ds(base + i * zr, zr)])
        return 0
    lax.fori_loop(0, rows // zr, zc, 0)


def _sc_degree(dst2d, nt, cw):
    @functools.partial(
        pl.kernel,
        out_type=jax.ShapeDtypeStruct((NW, 1, nt), jnp.float32),
        mesh=_mesh(),
        compiler_params=pltpu.CompilerParams(needs_layout_passes=False, use_tc_tiling_on_sc=False),
        scratch_types=[
            pltpu.VMEM((cw, K), jnp.int32),
            pltpu.VMEM((nt,), jnp.float32),
        ],
    )
    def k(dst_hbm, out_hbm, dbuf, deg):
        c = lax.axis_index("c")
        s = lax.axis_index("s")
        wid = s * NC + c

        def zero(i, _):
            deg[pl.ds(i * NL, NL)] = jnp.zeros((NL,), jnp.float32)
            return 0
        lax.fori_loop(0, nt // NL, zero, 0)

        pltpu.sync_copy(dst_hbm.at[wid], dbuf)
        ones = jnp.ones((NL,), jnp.float32)

        def row(j, _):
            for t in range(K // NL):
                idx = dbuf[j, pl.ds(t * NL, NL)]
                plsc.addupdate_scatter(deg, [idx], ones)
            return 0
        lax.fori_loop(0, cw, row, 0)

        pltpu.sync_copy(deg, out_hbm.at[wid, 0])

    return k(dst2d)


def _sc_gcn(table, src2d, dst2d, nt, cw):
    d = table.shape[1]

    @functools.partial(
        pl.kernel,
        out_type=jax.ShapeDtypeStruct((NC, nt, d), jnp.float32),
        mesh=_mesh(),
        compiler_params=pltpu.CompilerParams(needs_layout_passes=False, use_tc_tiling_on_sc=False),
        scratch_types=[
            pltpu.VMEM_SHARED((nt, d), jnp.float32),
            pltpu.VMEM((cw, K), jnp.int32),
            pltpu.VMEM((cw, K), jnp.int32),
            pltpu.VMEM((K, d), jnp.float32),
            pltpu.VMEM((ZR, d), jnp.float32),
            pltpu.SemaphoreType.DMA,
        ],
    )
    def k(tab_hbm, src_hbm, dst_hbm, out_hbm, acc, sbuf, dbuf, rows, zbuf, sem):
        c = lax.axis_index("c")
        s = lax.axis_index("s")
        wid = s * NC + c

        _zero_rows(zbuf, ZR, d)
        _zero_shared(acc, zbuf, s, nt, d)
        plsc.subcore_barrier()

        pltpu.sync_copy(src_hbm.at[wid], sbuf)
        pltpu.sync_copy(dst_hbm.at[wid], dbuf)

        def chunk(j, _):
            pltpu.async_copy(tab_hbm.at[sbuf.at[j]], rows, sem).wait()
            pltpu.sync_copy(rows, acc.at[dbuf.at[j]], add=True)
            return 0
        lax.fori_loop(0, cw, chunk, 0)

        plsc.subcore_barrier()
        rpt = nt // NS
        pltpu.sync_copy(acc.at[pl.ds(s * rpt, rpt)],
                        out_hbm.at[c, pl.ds(s * rpt, rpt)])

    return k(table, src2d, dst2d)


def _sc_gat(hg, asrc, adst, src2d, dst2d, nt, cw):
    d = hg.shape[1]
    nh = d // NL

    @functools.partial(
        pl.kernel,
        out_type=[
            jax.ShapeDtypeStruct((NC, nt, d), jnp.float32),
            jax.ShapeDtypeStruct((NC, nt, NL), jnp.float32),
        ],
        mesh=_mesh(),
        compiler_params=pltpu.CompilerParams(needs_layout_passes=False, use_tc_tiling_on_sc=False),
        scratch_types=[
            pltpu.VMEM_SHARED((nt, d), jnp.float32),
            pltpu.VMEM_SHARED((nt, NL), jnp.float32),
            pltpu.VMEM((1, K), jnp.int32),
            pltpu.VMEM((1, K), jnp.int32),
            pltpu.VMEM((K, d), jnp.float32),
            pltpu.VMEM((K, NL), jnp.float32),
            pltpu.VMEM((K, NL), jnp.float32),
            pltpu.VMEM((K, NL), jnp.float32),
            pltpu.VMEM((ZR, d), jnp.float32),
            pltpu.SemaphoreType.DMA,
        ],
    )
    def k(hg_hbm, as_hbm, ad_hbm, src_hbm, dst_hbm, outa_hbm, outs_hbm,
          acca, accs, sbuf, dbuf, rows, ea, eb, wbuf, zbuf, sem):
        c = lax.axis_index("c")
        s = lax.axis_index("s")
        wid = s * NC + c

        _zero_rows(zbuf, ZR, d)
        _zero_shared(acca, zbuf, s, nt, d)
        def zw(j, _):
            wbuf[j, pl.ds(0, NL)] = jnp.zeros((NL,), jnp.float32)
            return 0
        lax.fori_loop(0, K, zw, 0)
        rows_s = nt // NS
        base_s = s * rows_s
        def zs(i, _):
            pltpu.sync_copy(wbuf, accs.at[pl.ds(base_s + i * K, K)])
            return 0
        lax.fori_loop(0, rows_s // K, zs, 0)
        plsc.subcore_barrier()

        def chunk(j, _):
            pltpu.sync_copy(src_hbm.at[wid, j], sbuf.at[0])
            pltpu.sync_copy(dst_hbm.at[wid, j], dbuf.at[0])
            pltpu.async_copy(as_hbm.at[sbuf.at[0]], ea, sem).wait()
            pltpu.async_copy(ad_hbm.at[dbuf.at[0]], eb, sem).wait()
            pltpu.async_copy(hg_hbm.at[sbuf.at[0]], rows, sem).wait()

            def edge(i, _):
                e = ea[i, :] + eb[i, :]
                e = jnp.maximum(e, e * 0.2)
                w = jnp.exp(e)
                wbuf[i, :] = w
                for h in range(nh):
                    wh = plsc.load_gather(
                        wbuf,
                        [jnp.full((NL,), i, jnp.int32),
                         jnp.full((NL,), h, jnp.int32)])
                    rows[i, pl.ds(h * NL, NL)] = rows[i, pl.ds(h * NL, NL)] * wh
                return 0
            lax.fori_loop(0, K, edge, 0)

            pltpu.sync_copy(wbuf, accs.at[dbuf.at[0]], add=True)
            pltpu.sync_copy(rows, acca.at[dbuf.at[0]], add=True)
            return 0
        lax.fori_loop(0, cw, chunk, 0)

        plsc.subcore_barrier()
        rpt = nt // NS
        pltpu.sync_copy(acca.at[pl.ds(s * rpt, rpt)],
                        outa_hbm.at[c, pl.ds(s * rpt, rpt)])
        pltpu.sync_copy(accs.at[pl.ds(s * rpt, rpt)],
                        outs_hbm.at[c, pl.ds(s * rpt, rpt)])

    return k(hg, asrc, adst, src2d, dst2d)


BLK = 1024


def _dis_block(degp):
    deg = jnp.sum(degp, axis=0)
    return jnp.where(deg > 0, lax.rsqrt(deg), 0.0)


def _tc_scale_matmul(xp, w, degp, nt):
    d = xp.shape[1]
    do = w.shape[1]

    def body(x_ref, w_ref, deg_ref, o_ref):
        dis = _dis_block(deg_ref[...])
        o_ref[...] = jnp.dot(x_ref[...], w_ref[...],
                             preferred_element_type=jnp.float32) * dis[:, None]

    return pl.pallas_call(
        body,
        grid=(nt // BLK,),
        in_specs=[
            pl.BlockSpec((BLK, d), lambda i: (i, 0)),
            pl.BlockSpec((d, do), lambda i: (0, 0)),
            pl.BlockSpec((NW, BLK), lambda i: (0, i)),
        ],
        out_specs=pl.BlockSpec((BLK, do), lambda i: (i, 0)),
        out_shape=jax.ShapeDtypeStruct((nt, do), jnp.float32),
    )(xp, w, degp)


def _tc_mid(a1p, degp, b1, wg, mboth, nt):
    d = wg.shape[0]
    do = wg.shape[1]

    def body(a_ref, deg_ref, b_ref, wg_ref, mb_ref, hg_ref, as_ref, ad_ref):
        dis = _dis_block(deg_ref[...])
        a = a_ref[0] + a_ref[1]
        h1 = jnp.maximum(a * dis[:, None] + b_ref[...], 0.0)
        hg = jnp.dot(h1, wg_ref[...], preferred_element_type=jnp.float32)
        hg_ref[...] = hg
        aa = jnp.dot(hg, mb_ref[...], preferred_element_type=jnp.float32)
        as_ref[...] = aa[:, :NL]
        ad_ref[...] = aa[:, NL:]

    return pl.pallas_call(
        body,
        grid=(nt // BLK,),
        in_specs=[
            pl.BlockSpec((NC, BLK, d), lambda i: (0, i, 0)),
            pl.BlockSpec((NW, BLK), lambda i: (0, i)),
            pl.BlockSpec((1, d), lambda i: (0, 0)),
            pl.BlockSpec((d, do), lambda i: (0, 0)),
            pl.BlockSpec((do, 2 * NL), lambda i: (0, 0)),
        ],
        out_specs=[
            pl.BlockSpec((BLK, do), lambda i: (i, 0)),
            pl.BlockSpec((BLK, NL), lambda i: (i, 0)),
            pl.BlockSpec((BLK, NL), lambda i: (i, 0)),
        ],
        out_shape=[
            jax.ShapeDtypeStruct((nt, do), jnp.float32),
            jax.ShapeDtypeStruct((nt, NL), jnp.float32),
            jax.ShapeDtypeStruct((nt, NL), jnp.float32),
        ],
    )(a1p, degp, b1, wg, mboth)


def _tc_gat_out(a2p, sp, degp, bg, w2, pexp, nt):
    d = w2.shape[0]
    do = w2.shape[1]

    def body(a_ref, s_ref, deg_ref, b_ref, w2_ref, p_ref, o_ref):
        dis = _dis_block(deg_ref[...])
        a = a_ref[0] + a_ref[1]
        sv = s_ref[0] + s_ref[1]
        sexp = jnp.dot(sv, p_ref[...], preferred_element_type=jnp.float32)
        h2 = jnp.maximum(a / (sexp + 1e-16) + b_ref[...], 0.0)
        o_ref[...] = jnp.dot(h2, w2_ref[...],
                             preferred_element_type=jnp.float32) * dis[:, None]

    return pl.pallas_call(
        body,
        grid=(nt // BLK,),
        in_specs=[
            pl.BlockSpec((NC, BLK, d), lambda i: (0, i, 0)),
            pl.BlockSpec((NC, BLK, NL), lambda i: (0, i, 0)),
            pl.BlockSpec((NW, BLK), lambda i: (0, i)),
            pl.BlockSpec((1, d), lambda i: (0, 0)),
            pl.BlockSpec((d, do), lambda i: (0, 0)),
            pl.BlockSpec((NL, d), lambda i: (0, 0)),
        ],
        out_specs=pl.BlockSpec((BLK, do), lambda i: (i, 0)),
        out_shape=jax.ShapeDtypeStruct((nt, do), jnp.float32),
    )(a2p, sp, degp, bg, w2, pexp)


def _tc_final(a3p, degp, b2, nt):
    d = a3p.shape[2]

    def body(a_ref, deg_ref, b_ref, o_ref):
        dis = _dis_block(deg_ref[...])
        a = a_ref[0] + a_ref[1]
        o_ref[...] = a * dis[:, None] + b_ref[...]

    return pl.pallas_call(
        body,
        grid=(nt // BLK,),
        in_specs=[
            pl.BlockSpec((NC, BLK, d), lambda i: (0, i, 0)),
            pl.BlockSpec((NW, BLK), lambda i: (0, i)),
            pl.BlockSpec((1, d), lambda i: (0, 0)),
        ],
        out_specs=pl.BlockSpec((BLK, d), lambda i: (i, 0)),
        out_shape=jax.ShapeDtypeStruct((nt, d), jnp.float32),
    )(a3p, degp, b2)


def kernel(x, edge_index, W1, b1, Wg, att_src, att_dst, bg, W2, b2):
    n, d = x.shape
    e = edge_index.shape[1]
    h = att_src.shape[1]
    ch = att_src.shape[2]
    nt = ((n + 1 + BLK - 1) // BLK) * BLK

    loop = jnp.arange(n, dtype=jnp.int32)
    src = jnp.concatenate([edge_index[0].astype(jnp.int32), loop])
    dst = jnp.concatenate([edge_index[1].astype(jnp.int32), loop])
    etot = e + n
    cw = -(-etot // (NW * K))
    epad = NW * K * cw
    fill = jnp.full((epad - etot,), n, jnp.int32)
    src2d = jnp.concatenate([src, fill]).reshape(NW, cw, K)
    dst2d = jnp.concatenate([dst, fill]).reshape(NW, cw, K)

    sel = (jnp.arange(NL)[None, :] == (jnp.arange(h * ch) // ch)[:, None])
    sel = sel.astype(jnp.float32)
    msrc = att_src.reshape(h * ch)[:, None] * sel
    mdst = att_dst.reshape(h * ch)[:, None] * sel
    mboth = jnp.concatenate([msrc, mdst], axis=1)
    pexp = sel.T.copy()

    xp = jnp.pad(x, ((0, nt - n), (0, 0)))
    b1r = b1.reshape(1, d)
    bgr = bg.reshape(1, h * ch)
    b2r = b2.reshape(1, d)

    degp = _sc_degree(dst2d, nt, cw).reshape(NW, nt)
    g1 = _tc_scale_matmul(xp, W1, degp, nt)
    a1p = _sc_gcn(g1, src2d, dst2d, nt, cw)
    hg, asrc, adst = _tc_mid(a1p, degp, b1r, Wg, mboth, nt)
    a2p, sp = _sc_gat(hg, asrc, adst, src2d, dst2d, nt, cw)
    g2 = _tc_gat_out(a2p, sp, degp, bgr, W2, pexp, nt)
    a3p = _sc_gcn(g2, src2d, dst2d, nt, cw)
    out = _tc_final(a3p, degp, b2r, nt)
    return out[:n]


# speedup vs baseline: 59.8667x; 1.9171x over previous
import functools

import jax
import jax.numpy as jnp
from jax import lax
from jax.experimental import pallas as pl
from jax.experimental.pallas import tpu as pltpu
from jax.experimental.pallas import tpu_sc as plsc

NC = 2
NS = 16
NL = 16
NW = NC * NS
K = 96


def _mesh():
    return plsc.VectorSubcoreMesh(core_axis_name="c", subcore_axis_name="s")


ZR = 8


def _zero_rows(zbuf, nrow, ncol):
    def zrow(j, _):
        for t in range(ncol // NL):
            zbuf[j, pl.ds(t * NL, NL)] = jnp.zeros((NL,), jnp.float32)
        return 0
    lax.fori_loop(0, nrow, zrow, 0)


def _zero_shared(acc, zbuf, s, nt, ncol):
    zr = zbuf.shape[0]
    rows = nt // NS
    base = s * rows
    def zc(i, _):
        pltpu.sync_copy(zbuf, acc.at[pl.ds(base + i * zr, zr)])
        return 0
    lax.fori_loop(0, rows // zr, zc, 0)


def _sc_degree(dst2d, nt, cw):
    @functools.partial(
        pl.kernel,
        out_type=jax.ShapeDtypeStruct((NW, 1, nt), jnp.float32),
        mesh=_mesh(),
        compiler_params=pltpu.CompilerParams(needs_layout_passes=False, use_tc_tiling_on_sc=False),
        scratch_types=[
            pltpu.VMEM((cw, K), jnp.int32),
            pltpu.VMEM((nt,), jnp.float32),
        ],
    )
    def k(dst_hbm, out_hbm, dbuf, deg):
        c = lax.axis_index("c")
        s = lax.axis_index("s")
        wid = s * NC + c

        def zero(i, _):
            deg[pl.ds(i * NL, NL)] = jnp.zeros((NL,), jnp.float32)
            return 0
        lax.fori_loop(0, nt // NL, zero, 0)

        pltpu.sync_copy(dst_hbm.at[wid], dbuf)
        ones = jnp.ones((NL,), jnp.float32)

        def row(j, _):
            for t in range(K // NL):
                idx = dbuf[j, pl.ds(t * NL, NL)]
                plsc.addupdate_scatter(deg, [idx], ones)
            return 0
        lax.fori_loop(0, cw, row, 0)

        pltpu.sync_copy(deg, out_hbm.at[wid, 0])

    return k(dst2d)


def _sc_gcn(table, src2d, dst2d, nt, cw):
    d = table.shape[1]

    @functools.partial(
        pl.kernel,
        out_type=jax.ShapeDtypeStruct((NC, nt, d), jnp.float32),
        mesh=_mesh(),
        compiler_params=pltpu.CompilerParams(needs_layout_passes=False, use_tc_tiling_on_sc=False),
        scratch_types=[
            pltpu.VMEM_SHARED((nt, d), jnp.float32),
            pltpu.VMEM((cw, K), jnp.int32),
            pltpu.VMEM((cw, K), jnp.int32),
            pltpu.VMEM((K, d), jnp.float32),
            pltpu.VMEM((K, d), jnp.float32),
            pltpu.VMEM((ZR, d), jnp.float32),
            pltpu.SemaphoreType.DMA,
            pltpu.SemaphoreType.DMA,
        ],
    )
    def k(tab_hbm, src_hbm, dst_hbm, out_hbm, acc, sbuf, dbuf, rows0, rows1,
          zbuf, sem0, sem1):
        c = lax.axis_index("c")
        s = lax.axis_index("s")
        wid = s * NC + c

        _zero_rows(zbuf, ZR, d)
        _zero_shared(acc, zbuf, s, nt, d)
        plsc.subcore_barrier()

        pltpu.sync_copy(src_hbm.at[wid], sbuf)
        pltpu.sync_copy(dst_hbm.at[wid], dbuf)
        pltpu.async_copy(tab_hbm.at[sbuf.at[0]], rows0, sem0)

        def stage(j, cur, semc, nxt, semn):
            pltpu.make_async_copy(tab_hbm.at[sbuf.at[j]], cur, semc).wait()

            @pl.when(j + 1 < cw)
            def _():
                pltpu.async_copy(tab_hbm.at[sbuf.at[j + 1]], nxt, semn)

            pltpu.sync_copy(cur, acc.at[dbuf.at[j]], add=True)

        def pair(jj, _):
            j0 = jj * 2
            stage(j0, rows0, sem0, rows1, sem1)
            stage(j0 + 1, rows1, sem1, rows0, sem0)
            return 0
        lax.fori_loop(0, cw // 2, pair, 0)

        plsc.subcore_barrier()
        rpt = nt // NS
        pltpu.sync_copy(acc.at[pl.ds(s * rpt, rpt)],
                        out_hbm.at[c, pl.ds(s * rpt, rpt)])

    return k(table, src2d, dst2d)


def _sc_gat(hg, asrc, adst, src2d, dst2d, nt, cw):
    d = hg.shape[1]
    nh = d // NL

    @functools.partial(
        pl.kernel,
        out_type=[
            jax.ShapeDtypeStruct((NC, nt, d), jnp.float32),
            jax.ShapeDtypeStruct((NC, nt, NL), jnp.float32),
        ],
        mesh=_mesh(),
        compiler_params=pltpu.CompilerParams(needs_layout_passes=False, use_tc_tiling_on_sc=False),
        scratch_types=[
            pltpu.VMEM_SHARED((nt, d), jnp.float32),
            pltpu.VMEM_SHARED((nt, NL), jnp.float32),
            pltpu.VMEM((3, K), jnp.int32),
            pltpu.VMEM((3, K), jnp.int32),
            pltpu.VMEM((2, K, d), jnp.float32),
            pltpu.VMEM((2, K, NL), jnp.float32),
            pltpu.VMEM((2, K, NL), jnp.float32),
            pltpu.VMEM((K, NL), jnp.float32),
            pltpu.VMEM((ZR, d), jnp.float32),
            pltpu.SemaphoreType.DMA,
            pltpu.SemaphoreType.DMA,
            pltpu.SemaphoreType.DMA,
        ],
    )
    def k(hg_hbm, as_hbm, ad_hbm, src_hbm, dst_hbm, outa_hbm, outs_hbm,
          acca, accs, sbuf, dbuf, rows, ea, eb, wbuf, zbuf,
          sem_i, sem_g0, sem_g1):
        c = lax.axis_index("c")
        s = lax.axis_index("s")
        wid = s * NC + c

        _zero_rows(zbuf, ZR, d)
        _zero_shared(acca, zbuf, s, nt, d)
        def zw(j, _):
            wbuf[j, pl.ds(0, NL)] = jnp.zeros((NL,), jnp.float32)
            return 0
        lax.fori_loop(0, K, zw, 0)
        rows_s = nt // NS
        base_s = s * rows_s
        def zs(i, _):
            pltpu.sync_copy(wbuf, accs.at[pl.ds(base_s + i * K, K)])
            return 0
        lax.fori_loop(0, rows_s // K, zs, 0)
        plsc.subcore_barrier()

        def issue_gathers(jm3, p, semg):
            pltpu.async_copy(as_hbm.at[sbuf.at[jm3]], ea.at[p], semg)
            pltpu.async_copy(ad_hbm.at[dbuf.at[jm3]], eb.at[p], semg)
            pltpu.async_copy(hg_hbm.at[sbuf.at[jm3]], rows.at[p], semg)

        def wait_gathers(jm3, p, semg):
            pltpu.make_async_copy(as_hbm.at[sbuf.at[jm3]], ea.at[p], semg).wait()
            pltpu.make_async_copy(ad_hbm.at[dbuf.at[jm3]], eb.at[p], semg).wait()
            pltpu.make_async_copy(hg_hbm.at[sbuf.at[jm3]], rows.at[p], semg).wait()

        pltpu.sync_copy(src_hbm.at[wid, 0], sbuf.at[0])
        pltpu.sync_copy(dst_hbm.at[wid, 0], dbuf.at[0])
        pltpu.async_copy(src_hbm.at[wid, 1], sbuf.at[1], sem_i)
        pltpu.async_copy(dst_hbm.at[wid, 1], dbuf.at[1], sem_i)
        issue_gathers(0, 0, sem_g0)

        def stage(j, p, semc, q, semn):
            jm3 = lax.rem(j, 3)
            jp1m3 = lax.rem(j + 1, 3)
            jp2m3 = lax.rem(j + 2, 3)
            wait_gathers(jm3, p, semc)

            @pl.when(j + 1 < cw)
            def _():
                pltpu.make_async_copy(
                    src_hbm.at[wid, j + 1], sbuf.at[jp1m3], sem_i).wait()
                pltpu.make_async_copy(
                    dst_hbm.at[wid, j + 1], dbuf.at[jp1m3], sem_i).wait()
                issue_gathers(jp1m3, q, semn)

            @pl.when(j + 2 < cw)
            def _():
                pltpu.async_copy(src_hbm.at[wid, j + 2], sbuf.at[jp2m3], sem_i)
                pltpu.async_copy(dst_hbm.at[wid, j + 2], dbuf.at[jp2m3], sem_i)

            @plsc.parallel_loop(0, K, unroll=4)
            def _(i):
                e = ea[p, i, :] + eb[p, i, :]
                e = jnp.maximum(e, e * 0.2)
                wbuf[i, :] = jnp.exp(e)

            @plsc.parallel_loop(0, K, unroll=2)
            def _(i):
                iv = jnp.full((NL,), i, jnp.int32)
                for h in range(nh):
                    wh = plsc.load_gather(
                        wbuf, [iv, jnp.full((NL,), h, jnp.int32)])
                    rows[p, i, pl.ds(h * NL, NL)] = (
                        rows[p, i, pl.ds(h * NL, NL)] * wh)

            pltpu.sync_copy(wbuf, accs.at[dbuf.at[jm3]], add=True)
            pltpu.sync_copy(rows.at[p], acca.at[dbuf.at[jm3]], add=True)

        def pair(jj, _):
            j0 = jj * 2
            stage(j0, 0, sem_g0, 1, sem_g1)
            stage(j0 + 1, 1, sem_g1, 0, sem_g0)
            return 0
        lax.fori_loop(0, cw // 2, pair, 0)

        plsc.subcore_barrier()
        rpt = nt // NS
        pltpu.sync_copy(acca.at[pl.ds(s * rpt, rpt)],
                        outa_hbm.at[c, pl.ds(s * rpt, rpt)])
        pltpu.sync_copy(accs.at[pl.ds(s * rpt, rpt)],
                        outs_hbm.at[c, pl.ds(s * rpt, rpt)])

    return k(hg, asrc, adst, src2d, dst2d)


BLK = 1024


def _dis_block(degp):
    deg = jnp.sum(degp, axis=0)
    return jnp.where(deg > 0, lax.rsqrt(deg), 0.0)


def _tc_scale_matmul(xp, w, degp, nt):
    d = xp.shape[1]
    do = w.shape[1]

    def body(x_ref, w_ref, deg_ref, o_ref):
        dis = _dis_block(deg_ref[...])
        o_ref[...] = jnp.dot(x_ref[...], w_ref[...],
                             preferred_element_type=jnp.float32) * dis[:, None]

    return pl.pallas_call(
        body,
        grid=(nt // BLK,),
        in_specs=[
            pl.BlockSpec((BLK, d), lambda i: (i, 0)),
            pl.BlockSpec((d, do), lambda i: (0, 0)),
            pl.BlockSpec((NW, BLK), lambda i: (0, i)),
        ],
        out_specs=pl.BlockSpec((BLK, do), lambda i: (i, 0)),
        out_shape=jax.ShapeDtypeStruct((nt, do), jnp.float32),
    )(xp, w, degp)


def _tc_mid(a1p, degp, b1, wg, mboth, nt):
    d = wg.shape[0]
    do = wg.shape[1]

    def body(a_ref, deg_ref, b_ref, wg_ref, mb_ref, hg_ref, as_ref, ad_ref):
        dis = _dis_block(deg_ref[...])
        a = a_ref[0] + a_ref[1]
        h1 = jnp.maximum(a * dis[:, None] + b_ref[...], 0.0)
        hg = jnp.dot(h1, wg_ref[...], preferred_element_type=jnp.float32)
        hg_ref[...] = hg
        aa = jnp.dot(hg, mb_ref[...], preferred_element_type=jnp.float32)
        as_ref[...] = aa[:, :NL]
        ad_ref[...] = aa[:, NL:]

    return pl.pallas_call(
        body,
        grid=(nt // BLK,),
        in_specs=[
            pl.BlockSpec((NC, BLK, d), lambda i: (0, i, 0)),
            pl.BlockSpec((NW, BLK), lambda i: (0, i)),
            pl.BlockSpec((1, d), lambda i: (0, 0)),
            pl.BlockSpec((d, do), lambda i: (0, 0)),
            pl.BlockSpec((do, 2 * NL), lambda i: (0, 0)),
        ],
        out_specs=[
            pl.BlockSpec((BLK, do), lambda i: (i, 0)),
            pl.BlockSpec((BLK, NL), lambda i: (i, 0)),
            pl.BlockSpec((BLK, NL), lambda i: (i, 0)),
        ],
        out_shape=[
            jax.ShapeDtypeStruct((nt, do), jnp.float32),
            jax.ShapeDtypeStruct((nt, NL), jnp.float32),
            jax.ShapeDtypeStruct((nt, NL), jnp.float32),
        ],
    )(a1p, degp, b1, wg, mboth)


def _tc_gat_out(a2p, sp, degp, bg, w2, pexp, nt):
    d = w2.shape[0]
    do = w2.shape[1]

    def body(a_ref, s_ref, deg_ref, b_ref, w2_ref, p_ref, o_ref):
        dis = _dis_block(deg_ref[...])
        a = a_ref[0] + a_ref[1]
        sv = s_ref[0] + s_ref[1]
        sexp = jnp.dot(sv, p_ref[...], preferred_element_type=jnp.float32)
        h2 = jnp.maximum(a / (sexp + 1e-16) + b_ref[...], 0.0)
        o_ref[...] = jnp.dot(h2, w2_ref[...],
                             preferred_element_type=jnp.float32) * dis[:, None]

    return pl.pallas_call(
        body,
        grid=(nt // BLK,),
        in_specs=[
            pl.BlockSpec((NC, BLK, d), lambda i: (0, i, 0)),
            pl.BlockSpec((NC, BLK, NL), lambda i: (0, i, 0)),
            pl.BlockSpec((NW, BLK), lambda i: (0, i)),
            pl.BlockSpec((1, d), lambda i: (0, 0)),
            pl.BlockSpec((d, do), lambda i: (0, 0)),
            pl.BlockSpec((NL, d), lambda i: (0, 0)),
        ],
        out_specs=pl.BlockSpec((BLK, do), lambda i: (i, 0)),
        out_shape=jax.ShapeDtypeStruct((nt, do), jnp.float32),
    )(a2p, sp, degp, bg, w2, pexp)


def _tc_final(a3p, degp, b2, nt):
    d = a3p.shape[2]

    def body(a_ref, deg_ref, b_ref, o_ref):
        dis = _dis_block(deg_ref[...])
        a = a_ref[0] + a_ref[1]
        o_ref[...] = a * dis[:, None] + b_ref[...]

    return pl.pallas_call(
        body,
        grid=(nt // BLK,),
        in_specs=[
            pl.BlockSpec((NC, BLK, d), lambda i: (0, i, 0)),
            pl.BlockSpec((NW, BLK), lambda i: (0, i)),
            pl.BlockSpec((1, d), lambda i: (0, 0)),
        ],
        out_specs=pl.BlockSpec((BLK, d), lambda i: (i, 0)),
        out_shape=jax.ShapeDtypeStruct((nt, d), jnp.float32),
    )(a3p, degp, b2)


def kernel(x, edge_index, W1, b1, Wg, att_src, att_dst, bg, W2, b2):
    n, d = x.shape
    e = edge_index.shape[1]
    h = att_src.shape[1]
    ch = att_src.shape[2]
    nt = ((n + 1 + BLK - 1) // BLK) * BLK

    loop = jnp.arange(n, dtype=jnp.int32)
    src = jnp.concatenate([edge_index[0].astype(jnp.int32), loop])
    dst = jnp.concatenate([edge_index[1].astype(jnp.int32), loop])
    etot = e + n
    cw = -(-etot // (NW * K))
    epad = NW * K * cw
    fill = jnp.full((epad - etot,), n, jnp.int32)
    src2d = jnp.concatenate([src, fill]).reshape(NW, cw, K)
    dst2d = jnp.concatenate([dst, fill]).reshape(NW, cw, K)

    sel = (jnp.arange(NL)[None, :] == (jnp.arange(h * ch) // ch)[:, None])
    sel = sel.astype(jnp.float32)
    msrc = att_src.reshape(h * ch)[:, None] * sel
    mdst = att_dst.reshape(h * ch)[:, None] * sel
    mboth = jnp.concatenate([msrc, mdst], axis=1)
    pexp = sel.T.copy()

    xp = jnp.pad(x, ((0, nt - n), (0, 0)))
    b1r = b1.reshape(1, d)
    bgr = bg.reshape(1, h * ch)
    b2r = b2.reshape(1, d)

    degp = _sc_degree(dst2d, nt, cw).reshape(NW, nt)
    g1 = _tc_scale_matmul(xp, W1, degp, nt)
    a1p = _sc_gcn(g1, src2d, dst2d, nt, cw)
    hg, asrc, adst = _tc_mid(a1p, degp, b1r, Wg, mboth, nt)
    a2p, sp = _sc_gat(hg, asrc, adst, src2d, dst2d, nt, cw)
    g2 = _tc_gat_out(a2p, sp, degp, bgr, W2, pexp, nt)
    a3p = _sc_gcn(g2, src2d, dst2d, nt, cw)
    out = _tc_final(a3p, degp, b2r, nt)
    return out[:n]
